# Initial kernel scaffold; baseline (speedup 1.0000x reference)
import jax
import jax.numpy as jnp
from jax.experimental import pallas as pl


def kernel(logits, uniform):
    raise NotImplementedError("write your pallas kernel here")


# speedup vs baseline: 2.0072x; 2.0072x over previous
import functools

import jax
import jax.numpy as jnp
from jax.experimental import pallas as pl


def _sample_concrete_block(logits_ref, uniform_ref, out_ref):
    eps = jnp.finfo(jnp.float32).eps
    L = logits_ref[:]
    Lmax = jnp.max(L, axis=-1, keepdims=True)
    expL = jnp.exp(2.0 * (L - Lmax))
    U = uniform_ref[:]
    u = jnp.clip(U, eps, 1.0)
    rw = 1.0 / jnp.log(u)
    e = expL[:, None, :] * (rw * rw)
    s = jnp.sum(e, axis=-1, keepdims=True)
    samples = e / s
    out_ref[:] = jnp.max(samples, axis=1)


@functools.partial(jax.jit, static_argnames=("interpret",))
def kernel(logits, uniform, interpret=False):
    B, D = logits.shape
    _, K, _ = uniform.shape
    BB = 8
    return pl.pallas_call(
        _sample_concrete_block,
        grid=(B // BB,),
        in_specs=[
            pl.BlockSpec((BB, D), lambda b: (b, 0)),
            pl.BlockSpec((BB, K, D), lambda b: (b, 0, 0)),
        ],
        out_specs=pl.BlockSpec((BB, D), lambda b: (b, 0)),
        out_shape=jax.ShapeDtypeStruct((B, D), jnp.float32),
        interpret=interpret,
    )(logits, uniform)
